# baseline (speedup 1.0000x reference)
import jax
import jax.numpy as jnp
from jax.experimental import pallas as pl
from jax.experimental.pallas import tpu as pltpu


def _gcn_kernel(x_ref, s_ref, w_ref, o_ref):
    agg = jnp.dot(s_ref[...], x_ref[...], preferred_element_type=jnp.float32)
    o_ref[...] = jnp.maximum(
        jnp.dot(agg, w_ref[...], preferred_element_type=jnp.float32), 0.0
    )


def _pick_tile(n, candidates):
    for c in candidates:
        if n % c == 0:
            return c
    return n


@jax.jit
def kernel(x, support, weight):
    n, d_in = x.shape
    d_out = weight.shape[1]

    bm = _pick_tile(n, (200, 128, 8, 1))
    m_steps = n // bm

    return pl.pallas_call(
        _gcn_kernel,
        grid=(m_steps,),
        in_specs=[
            pl.BlockSpec((n, d_in), lambda i: (0, 0)),
            pl.BlockSpec((bm, n), lambda i: (i, 0)),
            pl.BlockSpec((d_in, d_out), lambda i: (0, 0)),
        ],
        out_specs=pl.BlockSpec((bm, d_out), lambda i: (i, 0)),
        out_shape=jax.ShapeDtypeStruct((n, d_out), jnp.float32),
        compiler_params=pltpu.CompilerParams(
            dimension_semantics=("arbitrary",),
        ),
    )(x, support, weight)


# speedup vs baseline: 1.0188x; 1.0188x over previous
import jax
import jax.numpy as jnp
from jax.experimental import pallas as pl
from jax.experimental.pallas import tpu as pltpu


def _gcn_kernel(x_ref, s_ref, w_ref, o_ref):
    s16 = s_ref[...].astype(jnp.bfloat16)
    x16 = x_ref[...].astype(jnp.bfloat16)
    agg = jnp.dot(s16, x16, preferred_element_type=jnp.float32)
    o_ref[...] = jnp.maximum(
        jnp.dot(agg, w_ref[...], preferred_element_type=jnp.float32), 0.0
    )


def _pick_tile(n, candidates):
    for c in candidates:
        if n % c == 0:
            return c
    return n


@jax.jit
def kernel(x, support, weight):
    n, d_in = x.shape
    d_out = weight.shape[1]

    bm = _pick_tile(n, (400, 256, 200, 128, 8, 1))
    m_steps = n // bm

    return pl.pallas_call(
        _gcn_kernel,
        grid=(m_steps,),
        in_specs=[
            pl.BlockSpec((n, d_in), lambda i: (0, 0)),
            pl.BlockSpec((bm, n), lambda i: (i, 0)),
            pl.BlockSpec((d_in, d_out), lambda i: (0, 0)),
        ],
        out_specs=pl.BlockSpec((bm, d_out), lambda i: (i, 0)),
        out_shape=jax.ShapeDtypeStruct((n, d_out), jnp.float32),
        compiler_params=pltpu.CompilerParams(
            dimension_semantics=("arbitrary",),
            vmem_limit_bytes=128 * 1024 * 1024,
        ),
    )(x, support, weight)


# speedup vs baseline: 1.0199x; 1.0012x over previous
import jax
import jax.numpy as jnp
from jax.experimental import pallas as pl
from jax.experimental.pallas import tpu as pltpu


def _gcn_kernel(x_ref, s_ref, w_ref, o_ref):
    s16 = s_ref[...].astype(jnp.bfloat16)
    x16 = x_ref[...].astype(jnp.bfloat16)
    agg = jnp.dot(s16, x16, preferred_element_type=jnp.float32)
    o_ref[...] = jnp.maximum(
        jnp.dot(agg, w_ref[...], preferred_element_type=jnp.float32), 0.0
    )


def _pick_tile(n, candidates):
    for c in candidates:
        if n % c == 0:
            return c
    return n


@jax.jit
def kernel(x, support, weight):
    n, d_in = x.shape
    d_out = weight.shape[1]

    bm = _pick_tile(n, (400, 256, 200, 128, 8, 1))
    m_steps = n // bm

    return pl.pallas_call(
        _gcn_kernel,
        grid=(m_steps,),
        in_specs=[
            pl.BlockSpec((n, d_in), lambda i: (0, 0)),
            pl.BlockSpec((bm, n), lambda i: (i, 0)),
            pl.BlockSpec((d_in, d_out), lambda i: (0, 0)),
        ],
        out_specs=pl.BlockSpec((bm, d_out), lambda i: (i, 0)),
        out_shape=jax.ShapeDtypeStruct((n, d_out), jnp.float32),
        compiler_params=pltpu.CompilerParams(
            dimension_semantics=("parallel",),
            vmem_limit_bytes=128 * 1024 * 1024,
        ),
    )(x, support, weight)


# speedup vs baseline: 1.0212x; 1.0013x over previous
import jax
import jax.numpy as jnp
from jax.experimental import pallas as pl
from jax.experimental.pallas import tpu as pltpu


def _gcn_kernel(x_ref, s_ref, w_ref, o_ref):
    agg = jnp.dot(s_ref[...], x_ref[...], preferred_element_type=jnp.float32)
    o_ref[...] = jnp.maximum(
        jnp.dot(agg, w_ref[...], preferred_element_type=jnp.float32), 0.0
    )


def _pick_tile(n, candidates):
    for c in candidates:
        if n % c == 0:
            return c
    return n


@jax.jit
def kernel(x, support, weight):
    n, d_in = x.shape
    d_out = weight.shape[1]

    bm = _pick_tile(n, (400, 256, 200, 128, 8, 1))
    m_steps = n // bm

    return pl.pallas_call(
        _gcn_kernel,
        grid=(m_steps,),
        in_specs=[
            pl.BlockSpec((n, d_in), lambda i: (0, 0)),
            pl.BlockSpec((bm, n), lambda i: (i, 0)),
            pl.BlockSpec((d_in, d_out), lambda i: (0, 0)),
        ],
        out_specs=pl.BlockSpec((bm, d_out), lambda i: (i, 0)),
        out_shape=jax.ShapeDtypeStruct((n, d_out), jnp.float32),
        compiler_params=pltpu.CompilerParams(
            dimension_semantics=("parallel",),
            vmem_limit_bytes=128 * 1024 * 1024,
        ),
    )(x, support, weight)
